# Initial kernel scaffold; baseline (speedup 1.0000x reference)
import jax
import jax.numpy as jnp
from jax.experimental import pallas as pl


def kernel(x, edge_index, edge_weight, H, s_xz_W, s_xz_b, s_hz_W, s_hz_b, s_xr_W, s_xr_b, s_hr_W, s_hr_b, s_xh_W, s_xh_b, s_hh_W, s_hh_b, t_xz_W, t_xz_b, t_hz_W, t_hz_b, t_xr_W, t_xr_b, t_hr_W, t_hr_b, t_xh_W, t_xh_b, t_hh_W, t_hh_b, bn1_g, bn1_b, bn2_g, bn2_b, out_W, out_b):
    raise NotImplementedError("write your pallas kernel here")


# speedup vs baseline: 5.5750x; 5.5750x over previous
import functools

import jax
import jax.numpy as jnp
from jax import lax
from jax.experimental import pallas as pl
from jax.experimental.pallas import tpu as pltpu
from jax.experimental.pallas import tpu_sc as plsc

N = 10000
E = 320000
D_IN = 128
HID = 64
D_OUT = 128

NC = 2
NS = 16
NW = NC * NS
EPT = E // NW
CH = 80
NCHUNK = EPT // CH
NROWCH = N // CH

_MESH = plsc.VectorSubcoreMesh(core_axis_name="c", subcore_axis_name="s")

_F32 = jnp.float32
_LANES = 16


def _zeros16():
    return jnp.zeros((_LANES,), _F32)


@functools.partial(
    pl.kernel,
    out_type=jax.ShapeDtypeStruct((NC, N), _F32),
    mesh=_MESH,
    compiler_params=pltpu.CompilerParams(needs_layout_passes=False),
    scratch_types=[
        pltpu.VMEM((CH,), jnp.int32),
        pltpu.VMEM((CH,), _F32),
        pltpu.VMEM((2000,), _F32),
        pltpu.VMEM_SHARED((N,), _F32),
    ],
)
def _deg_kernel(src_hbm, w_hbm, out_hbm, idx_v, val_v, zb_v, acc_sh):
    c = lax.axis_index("c")
    s = lax.axis_index("s")
    wid = s * NC + c

    def zb(i, carry):
        zb_v[pl.ds(i * _LANES, _LANES)] = _zeros16()
        return carry

    lax.fori_loop(0, 2000 // _LANES, zb, 0)

    @pl.when(s == 0)
    def _():
        for k in range(N // 2000):
            pltpu.sync_copy(zb_v, acc_sh.at[pl.ds(k * 2000, 2000)])

    plsc.subcore_barrier()

    base = wid * EPT

    def chunk(t, carry):
        off = base + t * CH
        pltpu.sync_copy(src_hbm.at[pl.ds(off, CH)], idx_v)
        pltpu.sync_copy(w_hbm.at[pl.ds(off, CH)], val_v)
        pltpu.sync_copy(val_v, acc_sh.at[idx_v], add=True)
        return carry

    lax.fori_loop(0, NCHUNK, chunk, 0)
    plsc.subcore_barrier()

    @pl.when(s == 0)
    def _():
        pltpu.sync_copy(acc_sh, out_hbm.at[c])


def _dis_body(degp_ref, dis_ref):
    d = degp_ref[0:1, :] + degp_ref[1:2, :]
    mask = d > 0.0
    dis_ref[...] = jnp.where(mask, lax.rsqrt(jnp.where(mask, d, 1.0)), 0.0)


def _dis(degp):
    return pl.pallas_call(
        _dis_body,
        out_shape=jax.ShapeDtypeStruct((1, N), _F32),
    )(degp)


@functools.partial(
    pl.kernel,
    out_type=jax.ShapeDtypeStruct((E,), _F32),
    mesh=_MESH,
    compiler_params=pltpu.CompilerParams(needs_layout_passes=False),
    scratch_types=[
        pltpu.VMEM((N,), _F32),
        pltpu.VMEM((CH,), jnp.int32),
        pltpu.VMEM((CH,), jnp.int32),
        pltpu.VMEM((CH,), _F32),
        pltpu.VMEM((CH,), _F32),
    ],
)
def _nw_kernel(src_hbm, dst_hbm, w_hbm, dis_hbm, out_hbm,
               dis_v, si_v, di_v, wv_v, nw_v):
    c = lax.axis_index("c")
    s = lax.axis_index("s")
    wid = s * NC + c

    pltpu.sync_copy(dis_hbm, dis_v)

    base = wid * EPT

    def chunk(t, carry):
        off = base + t * CH
        pltpu.sync_copy(src_hbm.at[pl.ds(off, CH)], si_v)
        pltpu.sync_copy(dst_hbm.at[pl.ds(off, CH)], di_v)
        pltpu.sync_copy(w_hbm.at[pl.ds(off, CH)], wv_v)

        def inner(i, carry2):
            sl = pl.ds(i * _LANES, _LANES)
            dsrc = plsc.load_gather(dis_v, [si_v[sl]])
            ddst = plsc.load_gather(dis_v, [di_v[sl]])
            nw_v[sl] = dsrc * ddst * (-wv_v[sl])
            return carry2

        lax.fori_loop(0, CH // _LANES, inner, 0)
        pltpu.sync_copy(nw_v, out_hbm.at[pl.ds(off, CH)])
        return carry

    lax.fori_loop(0, NCHUNK, chunk, 0)


def _make_prop(fs):
    n_t = len(fs)
    out_type = tuple(jax.ShapeDtypeStruct((NC, N, f), _F32) for f in fs)
    scratch = [
        pltpu.VMEM((CH,), jnp.int32),
        pltpu.VMEM((CH,), jnp.int32),
        pltpu.VMEM((CH,), _F32),
    ]
    scratch += [pltpu.VMEM((CH, f), _F32) for f in fs]
    scratch += [pltpu.VMEM_SHARED((N, f), _F32) for f in fs]

    @functools.partial(
        pl.kernel, out_type=out_type, mesh=_MESH, scratch_types=scratch,
        compiler_params=pltpu.CompilerParams(needs_layout_passes=False,
                                             use_tc_tiling_on_sc=False))
    def prop(*refs):
        tabs = refs[:n_t]
        src_hbm, dst_hbm, nw_hbm = refs[n_t:n_t + 3]
        outs = refs[n_t + 3:n_t + 3 + n_t]
        si_v, di_v, wv_v = refs[2 * n_t + 3:2 * n_t + 6]
        rows = refs[2 * n_t + 6:3 * n_t + 6]
        accs = refs[3 * n_t + 6:]

        c = lax.axis_index("c")
        s = lax.axis_index("s")
        wid = s * NC + c

        def zrow(i, carry):
            for r in rows:
                for j in range(r.shape[1] // _LANES):
                    r[i, pl.ds(j * _LANES, _LANES)] = _zeros16()
            return carry

        lax.fori_loop(0, CH, zrow, 0)

        for k in range(8):
            ck = k * NS + s

            @pl.when(ck < NROWCH)
            def _():
                for r, a in zip(rows, accs):
                    pltpu.sync_copy(r, a.at[pl.ds(ck * CH, CH)])

        plsc.subcore_barrier()

        base = wid * EPT

        def chunk(t, carry):
            off = base + t * CH
            pltpu.sync_copy(src_hbm.at[pl.ds(off, CH)], si_v)
            pltpu.sync_copy(dst_hbm.at[pl.ds(off, CH)], di_v)
            pltpu.sync_copy(nw_hbm.at[pl.ds(off, CH)], wv_v)
            for tab, r in zip(tabs, rows):
                pltpu.sync_copy(tab.at[si_v], r)

            def scale(i, carry2):
                sv = plsc.load_gather(wv_v, [jnp.full((_LANES,), i, jnp.int32)])
                for r in rows:
                    for j in range(r.shape[1] // _LANES):
                        sl = pl.ds(j * _LANES, _LANES)
                        r[i, sl] = r[i, sl] * sv
                return carry2

            lax.fori_loop(0, CH, scale, 0)
            for r, a in zip(rows, accs):
                pltpu.sync_copy(r, a.at[di_v], add=True)
            return carry

        lax.fori_loop(0, NCHUNK, chunk, 0)
        plsc.subcore_barrier()

        for k in range(8):
            ck = k * NS + s

            @pl.when(ck < NROWCH)
            def _():
                for a, o in zip(accs, outs):
                    pltpu.sync_copy(a.at[pl.ds(ck * CH, CH)],
                                    o.at[c, pl.ds(ck * CH, CH)])

    return prop


_prop128 = _make_prop((D_OUT,))
_prop64 = _make_prop((HID,))


_R = 1000
_G = N // _R


def _mm(a, b):
    return jnp.dot(a, b, precision=lax.Precision.HIGHEST,
                   preferred_element_type=_F32)


def _full(shape):
    nd = len(shape)
    return pl.BlockSpec(shape, lambda i, _n=nd: (0,) * _n)


def _t2_body(x_ref, xp_ref, wz0, wz1, wh0, wh1, bxz, bhz, bxh, bhh,
             h1_ref, st_ref):
    pi = pl.program_id(0)
    xv = x_ref[...]
    xp = xp_ref[0] + xp_ref[1]
    az = _mm(xv, wz0[...]) + _mm(xp, wz1[...]) + bxz[...] + bhz[...]
    ah = _mm(xv, wh0[...]) + _mm(xp, wh1[...]) + bxh[...] + bhh[...]
    z = jax.nn.sigmoid(az)
    ht = jnp.tanh(ah)
    h1 = (1.0 - z) * ht
    h1_ref[...] = h1

    @pl.when(pi == 0)
    def _():
        st_ref[...] = jnp.zeros_like(st_ref)

    st_ref[...] += jnp.stack([jnp.sum(h1, axis=0), jnp.sum(h1 * h1, axis=0)])


def _t2(x, xp2, wz0, wz1, wh0, wh1, bxz, bhz, bxh, bhh):
    return pl.pallas_call(
        _t2_body,
        grid=(_G,),
        in_specs=[
            pl.BlockSpec((_R, D_IN), lambda i: (i, 0)),
            pl.BlockSpec((NC, _R, D_IN), lambda i: (0, i, 0)),
            _full((D_IN, HID)), _full((D_IN, HID)),
            _full((D_IN, HID)), _full((D_IN, HID)),
            _full((1, HID)), _full((1, HID)), _full((1, HID)), _full((1, HID)),
        ],
        out_specs=[
            pl.BlockSpec((_R, HID), lambda i: (i, 0)),
            pl.BlockSpec((2, HID), lambda i: (0, 0)),
        ],
        out_shape=[
            jax.ShapeDtypeStruct((N, HID), _F32),
            jax.ShapeDtypeStruct((2, HID), _F32),
        ],
    )(x, xp2, wz0, wz1, wh0, wh1, bxz, bhz, bxh, bhh)


def _bn_relu_body(a_ref, st_ref, g_ref, b_ref, o_ref):
    st = st_ref[...]
    m = st[0:1, :] / N
    v = st[1:2, :] / N - m * m
    inv = lax.rsqrt(v + 1e-5)
    o_ref[...] = jnp.maximum((a_ref[...] - m) * inv * g_ref[...] + b_ref[...],
                             0.0)


def _bn_relu(a, st, g, b, f):
    return pl.pallas_call(
        _bn_relu_body,
        grid=(_G,),
        in_specs=[
            pl.BlockSpec((_R, f), lambda i: (i, 0)),
            _full((2, f)), _full((1, f)), _full((1, f)),
        ],
        out_specs=pl.BlockSpec((_R, f), lambda i: (i, 0)),
        out_shape=jax.ShapeDtypeStruct((N, f), _F32),
    )(a, st, g, b)


def _t3_body(h_ref, hp_ref, H_ref, Hp_ref,
             wxz0, wxz1, whz0, whz1, wxr0, wxr1, whr0, whr1, wxh0, wxh1,
             bxz, bhz, bxr, bhr, bxh, bhh,
             z_ref, rh_ref, p_ref):
    hv = h_ref[...]
    hp = hp_ref[0] + hp_ref[1]
    Hv = H_ref[...]
    Hp = Hp_ref[0] + Hp_ref[1]
    az = (_mm(hv, wxz0[...]) + _mm(hp, wxz1[...]) +
          _mm(Hv, whz0[...]) + _mm(Hp, whz1[...]) + bxz[...] + bhz[...])
    ar = (_mm(hv, wxr0[...]) + _mm(hp, wxr1[...]) +
          _mm(Hv, whr0[...]) + _mm(Hp, whr1[...]) + bxr[...] + bhr[...])
    z = jax.nn.sigmoid(az)
    r = jax.nn.sigmoid(ar)
    z_ref[...] = z
    rh_ref[...] = r * Hv
    p_ref[...] = _mm(hv, wxh0[...]) + _mm(hp, wxh1[...]) + bxh[...] + bhh[...]


def _t3(h, hp2, H, Hp2, wxz0, wxz1, whz0, whz1, wxr0, wxr1, whr0, whr1,
        wxh0, wxh1, bxz, bhz, bxr, bhr, bxh, bhh):
    return pl.pallas_call(
        _t3_body,
        grid=(_G,),
        in_specs=[
            pl.BlockSpec((_R, HID), lambda i: (i, 0)),
            pl.BlockSpec((NC, _R, HID), lambda i: (0, i, 0)),
            pl.BlockSpec((_R, D_OUT), lambda i: (i, 0)),
            pl.BlockSpec((NC, _R, D_OUT), lambda i: (0, i, 0)),
            _full((HID, D_OUT)), _full((HID, D_OUT)),
            _full((D_OUT, D_OUT)), _full((D_OUT, D_OUT)),
            _full((HID, D_OUT)), _full((HID, D_OUT)),
            _full((D_OUT, D_OUT)), _full((D_OUT, D_OUT)),
            _full((HID, D_OUT)), _full((HID, D_OUT)),
            _full((1, D_OUT)), _full((1, D_OUT)), _full((1, D_OUT)),
            _full((1, D_OUT)), _full((1, D_OUT)), _full((1, D_OUT)),
        ],
        out_specs=[
            pl.BlockSpec((_R, D_OUT), lambda i: (i, 0)),
            pl.BlockSpec((_R, D_OUT), lambda i: (i, 0)),
            pl.BlockSpec((_R, D_OUT), lambda i: (i, 0)),
        ],
        out_shape=[
            jax.ShapeDtypeStruct((N, D_OUT), _F32),
            jax.ShapeDtypeStruct((N, D_OUT), _F32),
            jax.ShapeDtypeStruct((N, D_OUT), _F32),
        ],
    )(h, hp2, H, Hp2, wxz0, wxz1, whz0, whz1, wxr0, wxr1, whr0, whr1,
      wxh0, wxh1, bxz, bhz, bxr, bhr, bxh, bhh)


def _t4_body(z_ref, p_ref, rh_ref, rhp_ref, H_ref, whh0, whh1,
             H2_ref, st_ref):
    pi = pl.program_id(0)
    rhp = rhp_ref[0] + rhp_ref[1]
    ht = jnp.tanh(p_ref[...] + _mm(rh_ref[...], whh0[...]) +
                  _mm(rhp, whh1[...]))
    z = z_ref[...]
    h2 = z * H_ref[...] + (1.0 - z) * ht
    H2_ref[...] = h2

    @pl.when(pi == 0)
    def _():
        st_ref[...] = jnp.zeros_like(st_ref)

    st_ref[...] += jnp.stack([jnp.sum(h2, axis=0), jnp.sum(h2 * h2, axis=0)])


def _t4(z, p, rh, rhp2, H, whh0, whh1):
    return pl.pallas_call(
        _t4_body,
        grid=(_G,),
        in_specs=[
            pl.BlockSpec((_R, D_OUT), lambda i: (i, 0)),
            pl.BlockSpec((_R, D_OUT), lambda i: (i, 0)),
            pl.BlockSpec((_R, D_OUT), lambda i: (i, 0)),
            pl.BlockSpec((NC, _R, D_OUT), lambda i: (0, i, 0)),
            pl.BlockSpec((_R, D_OUT), lambda i: (i, 0)),
            _full((D_OUT, D_OUT)), _full((D_OUT, D_OUT)),
        ],
        out_specs=[
            pl.BlockSpec((_R, D_OUT), lambda i: (i, 0)),
            pl.BlockSpec((2, D_OUT), lambda i: (0, 0)),
        ],
        out_shape=[
            jax.ShapeDtypeStruct((N, D_OUT), _F32),
            jax.ShapeDtypeStruct((2, D_OUT), _F32),
        ],
    )(z, p, rh, rhp2, H, whh0, whh1)


def _t4b_body(a_ref, st_ref, g_ref, b_ref, w_ref, ob_ref, out_ref, h2_ref):
    st = st_ref[...]
    m = st[0:1, :] / N
    v = st[1:2, :] / N - m * m
    inv = lax.rsqrt(v + 1e-5)
    hb = jnp.maximum((a_ref[...] - m) * inv * g_ref[...] + b_ref[...], 0.0)
    h2_ref[...] = hb
    out_ref[...] = lax.dot_general(
        hb, w_ref[...], (((1,), (1,)), ((), ())),
        precision=lax.Precision.HIGHEST,
        preferred_element_type=_F32) + ob_ref[...]


def _t4b(a, st, g, b, w, ob):
    return pl.pallas_call(
        _t4b_body,
        grid=(_G,),
        in_specs=[
            pl.BlockSpec((_R, D_OUT), lambda i: (i, 0)),
            _full((2, D_OUT)), _full((1, D_OUT)), _full((1, D_OUT)),
            _full((D_OUT, D_OUT)), _full((1, D_OUT)),
        ],
        out_specs=[
            pl.BlockSpec((_R, D_OUT), lambda i: (i, 0)),
            pl.BlockSpec((_R, D_OUT), lambda i: (i, 0)),
        ],
        out_shape=[
            jax.ShapeDtypeStruct((N, D_OUT), _F32),
            jax.ShapeDtypeStruct((N, D_OUT), _F32),
        ],
    )(a, st, g, b, w, ob)


def kernel(x, edge_index, edge_weight, H,
           s_xz_W, s_xz_b, s_hz_W, s_hz_b, s_xr_W, s_xr_b, s_hr_W, s_hr_b,
           s_xh_W, s_xh_b, s_hh_W, s_hh_b,
           t_xz_W, t_xz_b, t_hz_W, t_hz_b, t_xr_W, t_xr_b, t_hr_W, t_hr_b,
           t_xh_W, t_xh_b, t_hh_W, t_hh_b,
           bn1_g, bn1_b, bn2_g, bn2_b, out_W, out_b):
    src = edge_index[0]
    dst = edge_index[1]

    def row(v):
        return v.reshape(1, -1)

    degp = _deg_kernel(src, edge_weight)
    dis = _dis(degp).reshape(N)
    nw = _nw_kernel(src, dst, edge_weight, dis)

    xp2, = _prop128(x, src, dst, nw)

    h1, st1 = _t2(x, xp2, s_xz_W[0], s_xz_W[1], s_xh_W[0], s_xh_W[1],
                  row(s_xz_b), row(s_hz_b), row(s_xh_b), row(s_hh_b))
    h = _bn_relu(h1, st1, row(bn1_g), row(bn1_b), HID)

    hp2, = _prop64(h, src, dst, nw)
    Hp2, = _prop128(H, src, dst, nw)

    z, rh, p = _t3(h, hp2, H, Hp2,
                   t_xz_W[0], t_xz_W[1], t_hz_W[0], t_hz_W[1],
                   t_xr_W[0], t_xr_W[1], t_hr_W[0], t_hr_W[1],
                   t_xh_W[0], t_xh_W[1],
                   row(t_xz_b), row(t_hz_b), row(t_xr_b), row(t_hr_b),
                   row(t_xh_b), row(t_hh_b))

    rhp2, = _prop128(rh, src, dst, nw)

    H2, st2 = _t4(z, p, rh, rhp2, H, t_hh_W[0], t_hh_W[1])
    out, h2 = _t4b(H2, st2, row(bn2_g), row(bn2_b), out_W, row(out_b))
    return out, h2
